# baseline (speedup 1.0000x reference)
import functools

import jax
import jax.numpy as jnp
from jax import lax
from jax.experimental import pallas as pl
from jax.experimental.pallas import tpu as pltpu
from jax.experimental.pallas import tpu_sc as plsc

N = 16384
C = 1000
NC = 2
NS = 16
L = 16
NW = NC * NS
RW = N // NW
CHUNK = 32
NCHUNK = RW // CHUNK
NGRP = CHUNK // L
UNROLL = 8


def _geo_body(cls_hbm, three_hbm, tr_hbm, sc_hbm, out_hbm,
              in_buf, tr_buf, sc_buf, three_buf, out_buf, sem0, sem1):
    cid = lax.axis_index("c")
    sid = lax.axis_index("s")
    wid = sid * NC + cid
    base = wid * RW

    pltpu.sync_copy(tr_hbm, tr_buf)
    pltpu.sync_copy(sc_hbm, sc_buf)
    pltpu.sync_copy(three_hbm.at[pl.ds(base, RW), :], three_buf)

    lanes = lax.iota(jnp.int32, L)
    sems = (sem0, sem1)

    def start(k):
        return pltpu.async_copy(
            cls_hbm.at[pl.ds(base + k * CHUNK, CHUNK), :],
            in_buf.at[k % 2], sems[k % 2])

    pending = start(0)
    for k in range(NCHUNK):
        nxt = start(k + 1) if k + 1 < NCHUNK else None
        pending.wait()
        grp_ref = in_buf.at[k % 2]
        for g in range(NGRP):
            rows = lanes + g * L

            def gbody(j, carry, rows=rows):
                best, bidx = carry
                c0 = j * UNROLL
                for u in range(UNROLL):
                    col = jnp.full((L,), c0 + u, jnp.int32)
                    v = plsc.load_gather(grp_ref, [rows, col])
                    m = v > best
                    best = jnp.where(m, v, best)
                    bidx = jnp.where(m, col, bidx)
                return best, bidx

            init = (jnp.full((L,), -jnp.inf, jnp.float32),
                    jnp.zeros((L,), jnp.int32))
            _, bidx = lax.fori_loop(0, C // UNROLL, gbody, init)

            rows_l = lanes + (k * CHUNK + g * L)
            for d in range(3):
                dd = jnp.full((L,), d, jnp.int32)
                tr = plsc.load_gather(tr_buf, [dd, bidx])
                sc = plsc.load_gather(sc_buf, [dd, bidx])
                th = plsc.load_gather(three_buf, [rows_l, dd])
                plsc.store_scatter(out_buf, [rows_l, dd], th * sc + tr)
        pending = nxt

    pltpu.sync_copy(out_buf, out_hbm.at[pl.ds(base, RW), :])


def kernel(class_pred, three_pred, geo_dict, translation, scale):
    del geo_dict
    mesh = plsc.VectorSubcoreMesh(core_axis_name="c", subcore_axis_name="s")
    f = functools.partial(
        pl.kernel,
        out_type=jax.ShapeDtypeStruct((N, 3), jnp.float32),
        mesh=mesh,
        scratch_types=[
            pltpu.VMEM((2, CHUNK, C), jnp.float32),
            pltpu.VMEM((3, C), jnp.float32),
            pltpu.VMEM((3, C), jnp.float32),
            pltpu.VMEM((RW, 3), jnp.float32),
            pltpu.VMEM((RW, 3), jnp.float32),
            pltpu.SemaphoreType.DMA,
            pltpu.SemaphoreType.DMA,
        ],
        compiler_params=pltpu.CompilerParams(
            use_tc_tiling_on_sc=False, needs_layout_passes=False),
    )(_geo_body)
    return f(class_pred, three_pred, translation, scale)
